# initial kernel scaffold (unmeasured)
import jax
import jax.numpy as jnp
from jax import lax
from jax.experimental import pallas as pl
from jax.experimental.pallas import tpu as pltpu

N_DEV = 8
M, NOUT = 4096, 2048
CH = M // N_DEV


def _ring_pos(i):
    return jnp.where(i < 4, i, 11 - i)


def _all_reduce(partial):
    def body(in_ref, out_ref, acc, stage, recvbuf,
             send_sems, recv_sems, load_sem, store_sem, credit_sem):
        my_id = lax.axis_index("i")
        r = _ring_pos(my_id)
        right_id = _ring_pos((r + 1) % N_DEV)
        left_id = _ring_pos((r + N_DEV - 1) % N_DEV)

        barrier = pltpu.get_barrier_semaphore()
        for nbr in (left_id, right_id):
            pl.semaphore_signal(barrier, inc=1, device_id=(nbr,),
                                device_id_type=pl.DeviceIdType.MESH)
        pl.semaphore_wait(barrier, 2)

        def send(src, slot):
            return pltpu.make_async_remote_copy(
                src_ref=src, dst_ref=recvbuf.at[slot],
                send_sem=send_sems.at[slot], recv_sem=recv_sems.at[slot],
                device_id=(right_id,), device_id_type=pl.DeviceIdType.MESH)

        def load(chunk, dst):
            return pltpu.make_async_copy(
                in_ref.at[pl.ds(chunk * CH, CH), :], dst, load_sem)

        def store(src, chunk):
            return pltpu.make_async_copy(
                src, out_ref.at[pl.ds(chunk * CH, CH), :], store_sem)

        def credit_to_left():
            pl.semaphore_signal(credit_sem, inc=1, device_id=(left_id,),
                                device_id_type=pl.DeviceIdType.MESH)

        cp = load(r, acc)
        cp.start()
        cp.wait()
        for s in range(N_DEV - 1):
            slot = s % 2
            rdma = send(acc, slot)
            if s >= 2:
                pl.semaphore_wait(credit_sem, 1)
            rdma.start()
            cp = load((r - s - 1) % N_DEV, stage)
            cp.start()
            rdma.wait()
            cp.wait()
            acc[...] = recvbuf[slot] + stage[...]
            credit_to_left()


        g = (r + 1) % N_DEV
        rdma = send(acc, 7 % 2)
        pl.semaphore_wait(credit_sem, 1)
        rdma.start()
        st = store(acc, g)
        st.start()
        rdma.wait()
        st.wait()

        for s in range(8, 14):
            slot = s % 2
            fslot = (s - 1) % 2
            cf = (r - (s - 8)) % N_DEV
            rdma = send(recvbuf.at[fslot], slot)
            pl.semaphore_wait(credit_sem, 1)
            rdma.start()
            st = store(recvbuf.at[fslot], cf)
            st.start()
            rdma.wait()
            st.wait()
            if s <= 12:
                credit_to_left()

        st = store(recvbuf.at[13 % 2], (r - 6) % N_DEV)
        st.start()
        st.wait()

    return pl.pallas_call(
        body,
        out_shape=jax.ShapeDtypeStruct((M, NOUT), jnp.float32),
        in_specs=[pl.BlockSpec(memory_space=pltpu.ANY)],
        out_specs=pl.BlockSpec(memory_space=pltpu.ANY),
        scratch_shapes=[
            pltpu.VMEM((CH, NOUT), jnp.float32),
            pltpu.VMEM((CH, NOUT), jnp.float32),
            pltpu.VMEM((2, CH, NOUT), jnp.float32),
            pltpu.SemaphoreType.DMA((2,)),
            pltpu.SemaphoreType.DMA((2,)),
            pltpu.SemaphoreType.DMA,
            pltpu.SemaphoreType.DMA,
            pltpu.SemaphoreType.REGULAR,
        ],
        compiler_params=pltpu.CompilerParams(collective_id=0),
    )(partial)


def kernel(x, w_mat):
    partial = jnp.dot(x, w_mat, preferred_element_type=jnp.float32)
    y = _all_reduce(partial)
    y = jnp.maximum(y, 0.0)
    amax = jnp.max(y)
    scale = jnp.maximum(amax, 1e-30) / 448.0
    q = (y / scale).astype(jnp.float8_e4m3fn).astype(jnp.float32) * scale
    return q


# baseline (device time: 769621 ns/iter reference)
import jax
import jax.numpy as jnp
from jax import lax
from jax.experimental import pallas as pl
from jax.experimental.pallas import tpu as pltpu

N_DEV = 8
M, NOUT = 4096, 2048
CH = M // N_DEV


def _ring_pos(i):
    return jnp.where(i < 4, i, 11 - i)


def _all_reduce(partial):
    def body(in_ref, out_ref, acc, stage, recvbuf,
             send_sems, recv_sems, load_sem, store_sem, credit_sem):
        my_id = lax.axis_index("i")
        r = _ring_pos(my_id)
        right_id = _ring_pos((r + 1) % N_DEV)
        left_id = _ring_pos((r + N_DEV - 1) % N_DEV)

        barrier = pltpu.get_barrier_semaphore()
        for nbr in (left_id, right_id):
            pl.semaphore_signal(barrier, inc=1, device_id=(nbr,),
                                device_id_type=pl.DeviceIdType.MESH)
        pl.semaphore_wait(barrier, 2)

        def send(src, slot):
            return pltpu.make_async_remote_copy(
                src_ref=src, dst_ref=recvbuf.at[slot],
                send_sem=send_sems.at[slot], recv_sem=recv_sems.at[slot],
                device_id=(right_id,), device_id_type=pl.DeviceIdType.MESH)

        def load(chunk, dst):
            return pltpu.make_async_copy(
                in_ref.at[pl.ds(chunk * CH, CH), :], dst, load_sem)

        def store(src, chunk):
            return pltpu.make_async_copy(
                src, out_ref.at[pl.ds(chunk * CH, CH), :], store_sem)

        def credit_to_left():
            pl.semaphore_signal(credit_sem, inc=1, device_id=(left_id,),
                                device_id_type=pl.DeviceIdType.MESH)

        cp = load(r, acc)
        cp.start()
        cp.wait()
        for s in range(N_DEV - 1):
            slot = s % 2
            rdma = send(acc, slot)
            if s >= 2:
                pl.semaphore_wait(credit_sem, 1)
            rdma.start()
            cp = load((r - s - 1) % N_DEV, stage)
            cp.start()
            rdma.wait()
            cp.wait()
            acc[...] = recvbuf[slot] + stage[...]
            credit_to_left()


        g = (r + 1) % N_DEV
        rdma = send(acc, 7 % 2)
        pl.semaphore_wait(credit_sem, 1)
        rdma.start()
        st = store(acc, g)
        st.start()
        rdma.wait()
        st.wait()

        for s in range(8, 14):
            slot = s % 2
            fslot = (s - 1) % 2
            cf = (r - (s - 8)) % N_DEV
            rdma = send(recvbuf.at[fslot], slot)
            pl.semaphore_wait(credit_sem, 1)
            rdma.start()
            st = store(recvbuf.at[fslot], cf)
            st.start()
            rdma.wait()
            st.wait()
            if s <= 12:
                credit_to_left()

        st = store(recvbuf.at[13 % 2], (r - 6) % N_DEV)
        st.start()
        st.wait()

    return pl.pallas_call(
        body,
        out_shape=jax.ShapeDtypeStruct((M, NOUT), jnp.float32),
        in_specs=[pl.BlockSpec(memory_space=pl.ANY)],
        out_specs=pl.BlockSpec(memory_space=pl.ANY),
        scratch_shapes=[
            pltpu.VMEM((CH, NOUT), jnp.float32),
            pltpu.VMEM((CH, NOUT), jnp.float32),
            pltpu.VMEM((2, CH, NOUT), jnp.float32),
            pltpu.SemaphoreType.DMA((2,)),
            pltpu.SemaphoreType.DMA((2,)),
            pltpu.SemaphoreType.DMA,
            pltpu.SemaphoreType.DMA,
            pltpu.SemaphoreType.REGULAR,
        ],
        compiler_params=pltpu.CompilerParams(collective_id=0),
    )(partial)


def kernel(x, w_mat):
    partial = jnp.dot(x, w_mat, preferred_element_type=jnp.float32,
                      precision=lax.Precision.HIGHEST)
    y = _all_reduce(partial)
    y = jnp.maximum(y, 0.0)
    amax = jnp.max(y)
    scale = jnp.maximum(amax, 1e-30) / 448.0
    q8 = (y / scale).astype(jnp.float8_e4m3fn)
    q8 = lax.optimization_barrier(q8)
    return q8.astype(jnp.float32) * scale


# device time: 456026 ns/iter; 1.6877x vs baseline; 1.6877x over previous
import jax
import jax.numpy as jnp
from jax import lax
from jax.experimental import pallas as pl
from jax.experimental.pallas import tpu as pltpu

N_DEV = 8
M, NOUT = 4096, 2048
CH = M // N_DEV
NH = NOUT // 2


def _ring_pos(i):
    return jnp.where(i < 4, i, 11 - i)


def _all_reduce(partial):
    def body(in_ref, out_ref, acc_a, acc_b, stage_a, stage_b, recv_a, recv_b,
             ss_a, rs_a, ss_b, rs_b, load_a, load_b, store_a, store_b,
             cred_a, cred_b):
        my_id = lax.axis_index("i")
        r = _ring_pos(my_id)
        right_id = _ring_pos((r + 1) % N_DEV)
        left_id = _ring_pos((r + N_DEV - 1) % N_DEV)

        barrier = pltpu.get_barrier_semaphore()
        for nbr in (left_id, right_id):
            pl.semaphore_signal(barrier, inc=1, device_id=(nbr,),
                                device_id_type=pl.DeviceIdType.MESH)
        pl.semaphore_wait(barrier, 2)

        rings = [
            dict(dst=right_id, up=left_id, coff=0, sgn=-1, acc=acc_a,
                 stage=stage_a, recv=recv_a, ss=ss_a, rs=rs_a,
                 load=load_a, store=store_a, cred=cred_a),
            dict(dst=left_id, up=right_id, coff=NH, sgn=1, acc=acc_b,
                 stage=stage_b, recv=recv_b, ss=ss_b, rs=rs_b,
                 load=load_b, store=store_b, cred=cred_b),
        ]

        def send(g, src, slot):
            return pltpu.make_async_remote_copy(
                src_ref=src, dst_ref=g["recv"].at[slot],
                send_sem=g["ss"].at[slot], recv_sem=g["rs"].at[slot],
                device_id=(g["dst"],), device_id_type=pl.DeviceIdType.MESH)

        def load(g, chunk, dst):
            return pltpu.make_async_copy(
                in_ref.at[pl.ds(chunk * CH, CH), pl.ds(g["coff"], NH)],
                dst, g["load"])

        def store(g, src, chunk):
            return pltpu.make_async_copy(
                src, out_ref.at[pl.ds(chunk * CH, CH), pl.ds(g["coff"], NH)],
                g["store"])

        def credit(g):
            pl.semaphore_signal(g["cred"], inc=1, device_id=(g["up"],),
                                device_id_type=pl.DeviceIdType.MESH)

        for g in rings:
            cp = load(g, r, g["acc"])
            cp.start()
            cp.wait()
        for s in range(N_DEV - 1):
            slot = s % 2
            rdmas = [send(g, g["acc"], slot) for g in rings]
            for g in rings:
                if s >= 2:
                    pl.semaphore_wait(g["cred"], 1)
            for rd in rdmas:
                rd.start()
            cps = [load(g, (r + g["sgn"] * (s + 1)) % N_DEV, g["stage"])
                   for g in rings]
            for cp in cps:
                cp.start()
            for rd in rdmas:
                rd.wait()
            for cp in cps:
                cp.wait()
            for g in rings:
                g["acc"][...] = g["recv"][slot] + g["stage"][...]
                credit(g)


        rdmas = [send(g, g["acc"], 7 % 2) for g in rings]
        for g in rings:
            pl.semaphore_wait(g["cred"], 1)
        for rd in rdmas:
            rd.start()
        sts = [store(g, g["acc"], (r - g["sgn"]) % N_DEV) for g in rings]
        for st in sts:
            st.start()
        for rd in rdmas:
            rd.wait()
        for st in sts:
            st.wait()

        for s in range(8, 14):
            slot = s % 2
            fslot = (s - 1) % 2
            rdmas = [send(g, g["recv"].at[fslot], slot) for g in rings]
            for g in rings:
                pl.semaphore_wait(g["cred"], 1)
            for rd in rdmas:
                rd.start()
            sts = [store(g, g["recv"].at[fslot],
                         (r + g["sgn"] * (s - 8)) % N_DEV) for g in rings]
            for st in sts:
                st.start()
            for rd in rdmas:
                rd.wait()
            for st in sts:
                st.wait()
            if s <= 12:
                for g in rings:
                    credit(g)

        sts = [store(g, g["recv"].at[13 % 2],
                     (r + g["sgn"] * 6) % N_DEV) for g in rings]
        for st in sts:
            st.start()
        for st in sts:
            st.wait()

    return pl.pallas_call(
        body,
        out_shape=jax.ShapeDtypeStruct((M, NOUT), jnp.float32),
        in_specs=[pl.BlockSpec(memory_space=pl.ANY)],
        out_specs=pl.BlockSpec(memory_space=pl.ANY),
        scratch_shapes=[
            pltpu.VMEM((CH, NH), jnp.float32),
            pltpu.VMEM((CH, NH), jnp.float32),
            pltpu.VMEM((CH, NH), jnp.float32),
            pltpu.VMEM((CH, NH), jnp.float32),
            pltpu.VMEM((2, CH, NH), jnp.float32),
            pltpu.VMEM((2, CH, NH), jnp.float32),
            pltpu.SemaphoreType.DMA((2,)),
            pltpu.SemaphoreType.DMA((2,)),
            pltpu.SemaphoreType.DMA((2,)),
            pltpu.SemaphoreType.DMA((2,)),
            pltpu.SemaphoreType.DMA,
            pltpu.SemaphoreType.DMA,
            pltpu.SemaphoreType.DMA,
            pltpu.SemaphoreType.DMA,
            pltpu.SemaphoreType.REGULAR,
            pltpu.SemaphoreType.REGULAR,
        ],
        compiler_params=pltpu.CompilerParams(collective_id=0),
    )(partial)


def kernel(x, w_mat):
    partial = jnp.dot(x, w_mat, preferred_element_type=jnp.float32,
                      precision=lax.Precision.HIGHEST)
    y = _all_reduce(partial)
    y = jnp.maximum(y, 0.0)
    amax = jnp.max(y)
    scale = jnp.maximum(amax, 1e-30) / 448.0
    q8 = (y / scale).astype(jnp.float8_e4m3fn)
    q8 = lax.optimization_barrier(q8)
    return q8.astype(jnp.float32) * scale


# device time: 386340 ns/iter; 1.9921x vs baseline; 1.1804x over previous
import jax
import jax.numpy as jnp
from jax import lax
from jax.experimental import pallas as pl
from jax.experimental.pallas import tpu as pltpu

N_DEV = 8
M, NOUT = 4096, 2048
CH = M // N_DEV
NH = NOUT // 2
NQ = NH // 2
LAST_CREDIT = 11


def _ring_pos(i):
    return jnp.where(i < 4, i, 11 - i)


def _all_reduce(partial):
    def body(in_ref, y_ref, amax_ref,
             acc_a, acc_b, stage_a, stage_b, recv_a, recv_b,
             ss_a, rs_a, ld_a, st_a, al_a, os_a,
             ss_b, rs_b, ld_b, st_b, al_b, os_b,
             cred_a, cred_b):
        my_id = lax.axis_index("i")
        r = _ring_pos(my_id)
        right_id = _ring_pos((r + 1) % N_DEV)
        left_id = _ring_pos((r + N_DEV - 1) % N_DEV)

        barrier = pltpu.get_barrier_semaphore()
        for nbr in (left_id, right_id):
            pl.semaphore_signal(barrier, inc=1, device_id=(nbr,),
                                device_id_type=pl.DeviceIdType.MESH)
        pl.semaphore_wait(barrier, 2)

        rings = [
            dict(dst=right_id, up=left_id, coff=0, sgn=-1, acc=acc_a,
                 stage=stage_a, recv=recv_a, ss=ss_a, rs=rs_a, ld=ld_a,
                 st=st_a, al=al_a, os=os_a, cred=cred_a),
            dict(dst=left_id, up=right_id, coff=NH, sgn=1, acc=acc_b,
                 stage=stage_b, recv=recv_b, ss=ss_b, rs=rs_b, ld=ld_b,
                 st=st_b, ld2=None, al=al_b, os=os_b, cred=cred_b),
        ]

        def ksl(k):
            return pl.ds(k * NQ, NQ)

        def rchunk(g, s):
            if s <= 6:
                return (r + g["sgn"] * (s + 1)) % N_DEV
            return (r + g["sgn"] * (s - 7)) % N_DEV

        def make_send(g, s, k, src):
            return pltpu.make_async_remote_copy(
                src_ref=src, dst_ref=g["recv"].at[s % 2, :, ksl(k)],
                send_sem=g["ss"].at[s % 2, k], recv_sem=g["rs"].at[s % 2, k],
                device_id=(g["dst"],), device_id_type=pl.DeviceIdType.MESH)

        def make_load(g, s, k):
            return pltpu.make_async_copy(
                in_ref.at[pl.ds(rchunk(g, s) * CH, CH),
                          pl.ds(g["coff"] + k * NQ, NQ)],
                g["stage"].at[s % 2, :, ksl(k)], g["ld"].at[s % 2, k])

        def make_store(g, s, k):
            return pltpu.make_async_copy(
                g["recv"].at[s % 2, :, ksl(k)],
                y_ref.at[pl.ds(rchunk(g, s) * CH, CH),
                         pl.ds(g["coff"] + k * NQ, NQ)],
                g["st"].at[s % 2, k])

        def credit(g):
            pl.semaphore_signal(g["cred"], inc=1, device_id=(g["up"],),
                                device_id_type=pl.DeviceIdType.MESH)

        m = jnp.float32(0.0)
        rd = {}
        st = {}
        own = {}

        accld = []
        for gi, g in enumerate(rings):
            cp = pltpu.make_async_copy(
                in_ref.at[pl.ds(r * CH, CH), pl.ds(g["coff"], NH)],
                g["acc"], g["al"])
            cp.start()
            accld.append(cp)
            for k in (0, 1):
                ld = make_load(g, 0, k)
                ld.start()
                rd[(gi, 0, k, "ld")] = ld
        for gi, g in enumerate(rings):
            accld[gi].wait()
            for k in (0, 1):
                d = make_send(g, 0, k, g["acc"].at[:, ksl(k)])
                rd[(gi, 0, k)] = d
                d.start()

        for s in range(14):
            for gi, g in enumerate(rings):
                for k in (0, 1):
                    d = rd[(gi, s, k)]
                    d.wait_send()
                    if 8 <= s:
                        st[(gi, s - 1, k)].wait()
                        if s - 1 <= LAST_CREDIT:
                            credit(g)
                    d.wait_recv()
                    if s <= 6:
                        rd[(gi, s, k, "ld")].wait()
                        if s + 1 <= 6:
                            ld = make_load(g, s + 1, k)
                            ld.start()
                            rd[(gi, s + 1, k, "ld")] = ld
                        g["acc"][:, ksl(k)] = (
                            g["recv"][s % 2, :, k * NQ:(k + 1) * NQ]
                            + g["stage"][s % 2, :, k * NQ:(k + 1) * NQ])
                        credit(g)
                        if s + 1 >= 2:
                            pl.semaphore_wait(g["cred"], 1)
                        d2 = make_send(g, s + 1, k, g["acc"].at[:, ksl(k)])
                        rd[(gi, s + 1, k)] = d2
                        d2.start()
                        if s == 6:
                            cp = pltpu.make_async_copy(
                                g["acc"].at[:, ksl(k)],
                                y_ref.at[pl.ds(((r - g["sgn"]) % N_DEV) * CH,
                                               CH),
                                         pl.ds(g["coff"] + k * NQ, NQ)],
                                g["os"].at[k])
                            cp.start()
                            own[(gi, k)] = cp
                            m = jnp.maximum(
                                m, jnp.max(g["acc"][:, k * NQ:(k + 1) * NQ]))
                    else:
                        cp = make_store(g, s, k)
                        cp.start()
                        st[(gi, s, k)] = cp
                        if s + 1 <= 13:
                            pl.semaphore_wait(g["cred"], 1)
                            d2 = make_send(g, s + 1, k,
                                           g["recv"].at[s % 2, :, ksl(k)])
                            rd[(gi, s + 1, k)] = d2
                            d2.start()
                        m = jnp.maximum(
                            m, jnp.max(g["recv"][s % 2, :,
                                                 k * NQ:(k + 1) * NQ]))

        for gi, g in enumerate(rings):
            for k in (0, 1):
                own[(gi, k)].wait()
                st[(gi, 13, k)].wait()
        amax_ref[0, 0] = m

    return pl.pallas_call(
        body,
        out_shape=[
            jax.ShapeDtypeStruct((M, NOUT), jnp.float32),
            jax.ShapeDtypeStruct((1, 1), jnp.float32),
        ],
        in_specs=[pl.BlockSpec(memory_space=pl.ANY)],
        out_specs=[pl.BlockSpec(memory_space=pl.ANY),
                   pl.BlockSpec(memory_space=pltpu.SMEM)],
        scratch_shapes=[
            pltpu.VMEM((CH, NH), jnp.float32),
            pltpu.VMEM((CH, NH), jnp.float32),
            pltpu.VMEM((2, CH, NH), jnp.float32),
            pltpu.VMEM((2, CH, NH), jnp.float32),
            pltpu.VMEM((2, CH, NH), jnp.float32),
            pltpu.VMEM((2, CH, NH), jnp.float32),
            pltpu.SemaphoreType.DMA((2, 2)),
            pltpu.SemaphoreType.DMA((2, 2)),
            pltpu.SemaphoreType.DMA((2, 2)),
            pltpu.SemaphoreType.DMA((2, 2)),
            pltpu.SemaphoreType.DMA,
            pltpu.SemaphoreType.DMA((2,)),
            pltpu.SemaphoreType.DMA((2, 2)),
            pltpu.SemaphoreType.DMA((2, 2)),
            pltpu.SemaphoreType.DMA((2, 2)),
            pltpu.SemaphoreType.DMA((2, 2)),
            pltpu.SemaphoreType.DMA,
            pltpu.SemaphoreType.DMA((2,)),
            pltpu.SemaphoreType.REGULAR,
            pltpu.SemaphoreType.REGULAR,
        ],
        compiler_params=pltpu.CompilerParams(collective_id=0),
    )(partial)


def kernel(x, w_mat):
    partial = jnp.dot(x, w_mat, preferred_element_type=jnp.float32,
                      precision=lax.Precision.HIGH)
    y, amax = _all_reduce(partial)
    scale = jnp.maximum(amax[0, 0], 1e-30) / 448.0
    q8 = (jnp.maximum(y, 0.0) / scale).astype(jnp.float8_e4m3fn)
    q8 = lax.optimization_barrier(q8)
    return q8.astype(jnp.float32) * scale
